# baseline (speedup 1.0000x reference)
import jax
import jax.numpy as jnp
from jax.experimental import pallas as pl
from jax.experimental.pallas import tpu as pltpu

_N = 4608
_C = 256
_BM = 256
_BN = 256
_NB = _N // _BM
_NCH = _N // _BN
_NCLS = 128


def _snnl_kernel(x_hbm, oh_hbm, yrow_ref, out_ref,
                 xbuf, ohbuf, acc_ref, sems):
    i = pl.program_id(0)
    i0 = pl.multiple_of(i * _BM, _BM)

    @pl.when(i == 0)
    def _load():
        cx = pltpu.make_async_copy(x_hbm, xbuf, sems.at[0])
        co = pltpu.make_async_copy(oh_hbm, ohbuf, sems.at[1])
        cx.start()
        co.start()
        cx.wait()
        co.wait()
        acc_ref[...] = jnp.zeros_like(acc_ref)

    xi = xbuf[pl.ds(i0, _BM), :]
    xi32 = xi.astype(jnp.float32)
    sq_i = jnp.sum(xi32 * xi32, axis=1, keepdims=True)
    rc = (jax.lax.broadcasted_iota(jnp.int32, (_BM, _BN), 0) + i0
          - jax.lax.broadcasted_iota(jnp.int32, (_BM, _BN), 1))
    ones8 = jnp.ones((8, _C), dtype=jnp.bfloat16)
    sacc = jnp.zeros((_BM, _NCLS), dtype=jnp.float32)
    for j in range(_NCH):
        j0 = j * _BN
        xj = xbuf[j0:j0 + _BN, :]
        dotv = jax.lax.dot_general(
            xi, xj, (((1,), (1,)), ((), ())),
            preferred_element_type=jnp.float32)
        sq_j = jax.lax.dot_general(
            ones8, xj * xj, (((1,), (1,)), ((), ())),
            preferred_element_type=jnp.float32)[0:1, :]
        d = jnp.maximum((sq_i + sq_j) - 2.0 * dotv, 0.0)
        e = jnp.where(rc == j0, 0.0, jnp.exp(-d)).astype(jnp.bfloat16)
        sacc = sacc + jax.lax.dot_general(
            e, ohbuf[j0:j0 + _BN, :], (((1,), (0,)), ((), ())),
            preferred_element_type=jnp.float32)
    den = jnp.sum(sacc, axis=1, keepdims=True)
    num = jnp.take_along_axis(sacc, yrow_ref[...], axis=1)
    r = jnp.log(num / den)
    part = jnp.sum(jnp.broadcast_to(r, (_BM, _NCLS)), axis=0, keepdims=True)
    acc_ref[...] += part

    @pl.when(i == _NB - 1)
    def _fin():
        out_ref[...] = acc_ref[0:1, 0:1] * (-1.0 / _N)


def kernel(labels, outputs, features, train_step, epoch):
    y = labels[:, ::8, ::8].reshape(-1).astype(jnp.int32)
    x = features.reshape(-1, _C).astype(jnp.bfloat16)
    oh = (y[:, None] == jnp.arange(_NCLS, dtype=jnp.int32)[None, :])
    oh = oh.astype(jnp.bfloat16)
    loss = pl.pallas_call(
        _snnl_kernel,
        grid=(_NB,),
        in_specs=[
            pl.BlockSpec(memory_space=pl.ANY),
            pl.BlockSpec(memory_space=pl.ANY),
            pl.BlockSpec((_BM, 1), lambda i: (i, 0)),
        ],
        out_specs=pl.BlockSpec((1, 1), lambda i: (0, 0)),
        out_shape=jax.ShapeDtypeStruct((1, 1), jnp.float32),
        scratch_shapes=[
            pltpu.VMEM((_N, _C), jnp.bfloat16),
            pltpu.VMEM((_N, _NCLS), jnp.bfloat16),
            pltpu.VMEM((1, _NCLS), jnp.float32),
            pltpu.SemaphoreType.DMA((2,)),
        ],
        compiler_params=pltpu.CompilerParams(
            dimension_semantics=("arbitrary",),
        ),
        name="snnl_fused",
    )(x, oh, y.reshape(_N, 1))
    return loss.reshape(())


# speedup vs baseline: 1.2874x; 1.2874x over previous
import jax
import jax.numpy as jnp
from jax.experimental import pallas as pl
from jax.experimental.pallas import tpu as pltpu

_N = 4608
_C = 256
_BM = 256
_BN = 512
_NB = _N // _BM
_NCH = _N // _BN
_NCLS = 128
_LANES = 128
_LOG2E = 1.4426950408889634
_TWO_LOG2E = 2.0 * _LOG2E


def _snnl_kernel(xt_hbm, oh_hbm, xi_ref, yrow_ref, out_ref,
                 xtbuf, ohbuf, estrip, acc_ref, sems):
    i = pl.program_id(0)
    i0 = pl.multiple_of(i * _BM, _BM)

    @pl.when(i == 0)
    def _load():
        cx = pltpu.make_async_copy(xt_hbm, xtbuf, sems.at[0])
        co = pltpu.make_async_copy(oh_hbm, ohbuf, sems.at[1])
        cx.start()
        co.start()
        cx.wait()
        co.wait()
        acc_ref[...] = jnp.zeros_like(acc_ref)

    xi = xi_ref[...]
    xi32 = xi.astype(jnp.float32)
    sqi_l = _LOG2E * jnp.sum(xi32 * xi32, axis=1, keepdims=True)
    l2e8 = jnp.full((8, _C), _LOG2E, dtype=jnp.bfloat16)
    for j in range(_NCH):
        j0 = j * _BN
        xtj = xtbuf[:, j0:j0 + _BN]
        dotv = jax.lax.dot_general(
            xi, xtj, (((1,), (0,)), ((), ())),
            preferred_element_type=jnp.float32)
        sqj_l = jax.lax.dot_general(
            l2e8, xtj * xtj, (((1,), (0,)), ((), ())),
            preferred_element_type=jnp.float32)[0:1, :]
        ssum = sqi_l + sqj_l
        for k in range(_BN // _LANES):
            s = slice(k * _LANES, (k + 1) * _LANES)
            e = jnp.exp2(_TWO_LOG2E * dotv[:, s] - ssum[:, s])
            estrip[:, j0 + k * _LANES:j0 + (k + 1) * _LANES] = (
                e.astype(jnp.bfloat16))
    rr = jax.lax.broadcasted_iota(jnp.int32, (_BM, _BM), 0)
    cc = jax.lax.broadcasted_iota(jnp.int32, (_BM, _BM), 1)
    dslice = pl.ds(i0, _BM)
    estrip[:, dslice] = jnp.where(
        rr == cc, jnp.bfloat16(0), estrip[:, dslice])
    sacc = jax.lax.dot_general(
        estrip[...], ohbuf[...], (((1,), (0,)), ((), ())),
        preferred_element_type=jnp.float32)
    den = jnp.sum(sacc, axis=1, keepdims=True)
    num = jnp.take_along_axis(sacc, yrow_ref[...], axis=1)
    r = jnp.log(num / den)
    part = jnp.sum(jnp.broadcast_to(r, (_BM, _NCLS)), axis=0, keepdims=True)
    acc_ref[...] += part

    @pl.when(i == _NB - 1)
    def _fin():
        out_ref[...] = acc_ref[0:1, 0:1] * (-1.0 / _N)


def kernel(labels, outputs, features, train_step, epoch):
    y = labels[:, ::8, ::8].reshape(-1).astype(jnp.int32)
    x = features.reshape(-1, _C).astype(jnp.bfloat16)
    xt = x.T
    oh = (y[:, None] == jnp.arange(_NCLS, dtype=jnp.int32)[None, :])
    oh = oh.astype(jnp.bfloat16)
    loss = pl.pallas_call(
        _snnl_kernel,
        grid=(_NB,),
        in_specs=[
            pl.BlockSpec(memory_space=pl.ANY),
            pl.BlockSpec(memory_space=pl.ANY),
            pl.BlockSpec((_BM, _C), lambda i: (i, 0)),
            pl.BlockSpec((_BM, 1), lambda i: (i, 0)),
        ],
        out_specs=pl.BlockSpec((1, 1), lambda i: (0, 0)),
        out_shape=jax.ShapeDtypeStruct((1, 1), jnp.float32),
        scratch_shapes=[
            pltpu.VMEM((_C, _N), jnp.bfloat16),
            pltpu.VMEM((_N, _NCLS), jnp.bfloat16),
            pltpu.VMEM((_BM, _N), jnp.bfloat16),
            pltpu.VMEM((1, _NCLS), jnp.float32),
            pltpu.SemaphoreType.DMA((2,)),
        ],
        compiler_params=pltpu.CompilerParams(
            dimension_semantics=("arbitrary",),
        ),
        name="snnl_fused",
    )(xt, oh, x, y.reshape(_N, 1))
    return loss.reshape(())


# speedup vs baseline: 1.3697x; 1.0639x over previous
import jax
import jax.numpy as jnp
from jax.experimental import pallas as pl
from jax.experimental.pallas import tpu as pltpu

_N = 4608
_C = 256
_BM = 256
_BN = 512
_NB = _N // _BM
_NCH = _N // _BN
_NCLS = 128
_LANES = 128
_LOG2E = 1.4426950408889634
_TWO_LOG2E = 2.0 * _LOG2E


def _snnl_kernel(xt_hbm, oh_hbm, xi_ref, yrow_ref, out_ref,
                 xtbuf, ohbuf, estrip, acc_ref, sems):
    i = pl.program_id(0)
    i0 = pl.multiple_of(i * _BM, _BM)

    @pl.when(i == 0)
    def _load():
        cx = pltpu.make_async_copy(xt_hbm, xtbuf, sems.at[0])
        co = pltpu.make_async_copy(oh_hbm, ohbuf, sems.at[1])
        cx.start()
        co.start()
        cx.wait()
        co.wait()
        acc_ref[...] = jnp.zeros_like(acc_ref)

    xi = xi_ref[...]
    l2e_col = jnp.full((_C, _NCLS), _LOG2E, dtype=jnp.bfloat16)
    sqi_l = jax.lax.dot_general(
        xi * xi, l2e_col, (((1,), (0,)), ((), ())),
        preferred_element_type=jnp.float32)
    l2e8 = jnp.full((8, _C), _LOG2E, dtype=jnp.bfloat16)
    for j in range(_NCH):
        j0 = j * _BN
        xtj = xtbuf[:, j0:j0 + _BN]
        dotv = jax.lax.dot_general(
            xi, xtj, (((1,), (0,)), ((), ())),
            preferred_element_type=jnp.float32)
        sqj_l = jax.lax.dot_general(
            l2e8, xtj * xtj, (((1,), (0,)), ((), ())),
            preferred_element_type=jnp.float32)[0:1, :]
        for k in range(_BN // _LANES):
            s = slice(k * _LANES, (k + 1) * _LANES)
            e = jnp.exp2(_TWO_LOG2E * dotv[:, s] - sqi_l - sqj_l[:, s])
            estrip[:, j0 + k * _LANES:j0 + (k + 1) * _LANES] = (
                e.astype(jnp.bfloat16))
    rr = jax.lax.broadcasted_iota(jnp.int32, (_BM, _BM), 0)
    cc = jax.lax.broadcasted_iota(jnp.int32, (_BM, _BM), 1)
    dslice = pl.ds(i0, _BM)
    estrip[:, dslice] = jnp.where(
        rr == cc, jnp.bfloat16(0), estrip[:, dslice])
    sacc = jax.lax.dot_general(
        estrip[...], ohbuf[...], (((1,), (0,)), ((), ())),
        preferred_element_type=jnp.float32)
    den = jnp.sum(sacc, axis=1, keepdims=True)
    num = jnp.take_along_axis(sacc, yrow_ref[...], axis=1)
    r = jnp.log(num / den)
    part = jnp.sum(jnp.broadcast_to(r, (_BM, _NCLS)), axis=0, keepdims=True)
    acc_ref[...] += part

    @pl.when(i == _NB - 1)
    def _fin():
        out_ref[...] = acc_ref[0:1, 0:1] * (-1.0 / _N)


def kernel(labels, outputs, features, train_step, epoch):
    y = labels[:, ::8, ::8].reshape(-1).astype(jnp.int32)
    x = features.reshape(-1, _C).astype(jnp.bfloat16)
    xt = x.T
    oh = (y[:, None] == jnp.arange(_NCLS, dtype=jnp.int32)[None, :])
    oh = oh.astype(jnp.bfloat16)
    loss = pl.pallas_call(
        _snnl_kernel,
        grid=(_NB,),
        in_specs=[
            pl.BlockSpec(memory_space=pl.ANY),
            pl.BlockSpec(memory_space=pl.ANY),
            pl.BlockSpec((_BM, _C), lambda i: (i, 0)),
            pl.BlockSpec((_BM, 1), lambda i: (i, 0)),
        ],
        out_specs=pl.BlockSpec((1, 1), lambda i: (0, 0)),
        out_shape=jax.ShapeDtypeStruct((1, 1), jnp.float32),
        scratch_shapes=[
            pltpu.VMEM((_C, _N), jnp.bfloat16),
            pltpu.VMEM((_N, _NCLS), jnp.bfloat16),
            pltpu.VMEM((_BM, _N), jnp.bfloat16),
            pltpu.VMEM((1, _NCLS), jnp.float32),
            pltpu.SemaphoreType.DMA((2,)),
        ],
        compiler_params=pltpu.CompilerParams(
            dimension_semantics=("arbitrary",),
        ),
        name="snnl_fused",
    )(xt, oh, x, y.reshape(_N, 1))
    return loss.reshape(())


# speedup vs baseline: 1.4165x; 1.0341x over previous
import jax
import jax.numpy as jnp
from jax.experimental import pallas as pl
from jax.experimental.pallas import tpu as pltpu

_N = 4608
_C = 256
_BM = 256
_BN = 512
_NB = _N // _BM
_NCH = _N // _BN
_NCLS = 128
_LANES = 128
_LOG2E = 1.4426950408889634
_TWO_LOG2E = 2.0 * _LOG2E


def _snnl_kernel(xt_hbm, oh_hbm, xi_ref, yrow_ref, out_ref,
                 xtbuf, ohbuf, estrip, acc_ref, sems):
    i = pl.program_id(0)
    i0 = pl.multiple_of(i * _BM, _BM)

    @pl.when(i == 0)
    def _load():
        cx = pltpu.make_async_copy(xt_hbm, xtbuf, sems.at[0])
        co = pltpu.make_async_copy(oh_hbm, ohbuf, sems.at[1])
        cx.start()
        co.start()
        cx.wait()
        co.wait()
        acc_ref[...] = jnp.zeros_like(acc_ref)

    xi = xi_ref[...]
    l2e_col = jnp.full((_C, _NCLS), _LOG2E, dtype=jnp.bfloat16)
    sqi_l = jax.lax.dot_general(
        xi * xi, l2e_col, (((1,), (0,)), ((), ())),
        preferred_element_type=jnp.float32)
    l2e8 = jnp.full((8, _C), _LOG2E, dtype=jnp.bfloat16)
    xi2 = xi * jnp.bfloat16(_TWO_LOG2E)
    for j in range(_NCH):
        j0 = j * _BN
        xtj = xtbuf[:, j0:j0 + _BN]
        dotv = jax.lax.dot_general(
            xi2, xtj, (((1,), (0,)), ((), ())),
            preferred_element_type=jnp.float32)
        sqj_l = jax.lax.dot_general(
            l2e8, xtj * xtj, (((1,), (0,)), ((), ())),
            preferred_element_type=jnp.float32)[0:1, :]
        for k in range(_BN // _LANES):
            s = slice(k * _LANES, (k + 1) * _LANES)
            e = jnp.exp2(dotv[:, s] - sqi_l - sqj_l[:, s])
            estrip[:, j0 + k * _LANES:j0 + (k + 1) * _LANES] = (
                e.astype(jnp.bfloat16))
    rr = jax.lax.broadcasted_iota(jnp.int32, (_BM, _BM), 0)
    cc = jax.lax.broadcasted_iota(jnp.int32, (_BM, _BM), 1)
    dslice = pl.ds(i0, _BM)
    estrip[:, dslice] = jnp.where(
        rr == cc, jnp.bfloat16(0), estrip[:, dslice])
    sacc = jax.lax.dot_general(
        estrip[...], ohbuf[...], (((1,), (0,)), ((), ())),
        preferred_element_type=jnp.float32)
    den = jnp.sum(sacc, axis=1, keepdims=True)
    num = jnp.take_along_axis(sacc, yrow_ref[...], axis=1)
    r = jnp.log(num / den)
    part = jnp.sum(jnp.broadcast_to(r, (_BM, _NCLS)), axis=0, keepdims=True)
    acc_ref[...] += part

    @pl.when(i == _NB - 1)
    def _fin():
        out_ref[...] = acc_ref[0:1, 0:1] * (-1.0 / _N)


def kernel(labels, outputs, features, train_step, epoch):
    y = labels[:, ::8, ::8].reshape(-1).astype(jnp.int32)
    x = features.reshape(-1, _C).astype(jnp.bfloat16)
    xt = x.T
    oh = (y[:, None] == jnp.arange(_NCLS, dtype=jnp.int32)[None, :])
    oh = oh.astype(jnp.bfloat16)
    loss = pl.pallas_call(
        _snnl_kernel,
        grid=(_NB,),
        in_specs=[
            pl.BlockSpec(memory_space=pl.ANY),
            pl.BlockSpec(memory_space=pl.ANY),
            pl.BlockSpec((_BM, _C), lambda i: (i, 0)),
            pl.BlockSpec((_BM, 1), lambda i: (i, 0)),
        ],
        out_specs=pl.BlockSpec((1, 1), lambda i: (0, 0)),
        out_shape=jax.ShapeDtypeStruct((1, 1), jnp.float32),
        scratch_shapes=[
            pltpu.VMEM((_C, _N), jnp.bfloat16),
            pltpu.VMEM((_N, _NCLS), jnp.bfloat16),
            pltpu.VMEM((_BM, _N), jnp.bfloat16),
            pltpu.VMEM((1, _NCLS), jnp.float32),
            pltpu.SemaphoreType.DMA((2,)),
        ],
        compiler_params=pltpu.CompilerParams(
            dimension_semantics=("arbitrary",),
        ),
        name="snnl_fused",
    )(xt, oh, x, y.reshape(_N, 1))
    return loss.reshape(())
